# baseline (speedup 1.0000x reference)
import functools

import jax
import jax.numpy as jnp
import numpy as np
from jax.experimental import pallas as pl
from jax.experimental.pallas import tpu as pltpu


def _block_kernel(x_ref, wr_ref, br_ref, wc_ref, bc_ref, w4_ref, b4_ref,
                  cmask_ref, pmask_ref, o_ref, *, H, W, n1, cr):
    xb = x_ref[0]

    r = jax.lax.dot_general(
        wr_ref[...], xb, (((1,), (0,)), ((), ())),
        preferred_element_type=jnp.float32) + br_ref[...]
    y1 = r[0:n1]
    tb = r[n1:].astype(jnp.bfloat16)

    pieces = []
    k = 0
    for dy in (-1, 0, 1):
        for dx in (-1, 0, 1):
            o = dy * W + dx
            if o == 0:
                pieces.append(tb)
            else:
                pieces.append(jnp.roll(tb, -o, axis=1) * cmask_ref[k])
            k += 1
    col = jnp.concatenate(pieces, axis=0)

    y23 = jax.lax.dot_general(
        wc_ref[...], col, (((1,), (0,)), ((), ())),
        preferred_element_type=jnp.float32) + bc_ref[...]

    mw = jnp.maximum(xb, jnp.maximum(
        jnp.roll(xb, 1, axis=1) + pmask_ref[0],
        jnp.roll(xb, -1, axis=1) + pmask_ref[1]))
    m = jnp.maximum(mw, jnp.maximum(
        jnp.roll(mw, W, axis=1) + pmask_ref[2],
        jnp.roll(mw, -W, axis=1) + pmask_ref[3]))
    y4 = jax.lax.dot_general(
        w4_ref[...], m, (((1,), (0,)), ((), ())),
        preferred_element_type=jnp.float32) + b4_ref[...]

    o_ref[0] = jnp.concatenate([y1, y23, y4], axis=0)


def _edge_masks(H, W):
    p = np.arange(H * W)
    hc, wc = p // W, p % W
    cmasks = []
    for dy in (-1, 0, 1):
        for dx in (-1, 0, 1):
            valid = np.ones(H * W, bool)
            if dy < 0:
                valid &= hc >= 1
            if dy > 0:
                valid &= hc <= H - 2
            if dx < 0:
                valid &= wc >= 1
            if dx > 0:
                valid &= wc <= W - 2
            cmasks.append(valid.astype(np.float32))
    cmask = np.stack(cmasks).reshape(9, 1, H * W)
    pmask = np.stack([
        np.where(wc >= 1, 0.0, -np.inf),
        np.where(wc <= W - 2, 0.0, -np.inf),
        np.where(hc >= 1, 0.0, -np.inf),
        np.where(hc <= H - 2, 0.0, -np.inf),
    ]).reshape(4, 1, H * W)
    return (jnp.asarray(cmask, jnp.bfloat16), jnp.asarray(pmask, jnp.bfloat16))


def kernel(x, w1, b1, w2r, b2r, w2, b2, w3r, b3r, w3, b3, w4, b4):
    N, Cin, H, W = x.shape
    HW = H * W
    n1 = w1.shape[1]
    n3r, n3 = w2.shape[2], w2.shape[3]
    n5r, n5 = w3.shape[2], w3.shape[3]
    pc = w4.shape[1]
    cr = n3r + n5r
    ctot = n1 + n3 + n5 + pc

    x2 = x.reshape(N, Cin, HW).astype(jnp.bfloat16)

    wrT = jnp.concatenate([w1, w2r, w3r], axis=1).T.astype(jnp.bfloat16)
    brA = jnp.concatenate([b1, b2r, b3r]).reshape(-1, 1)
    blocks = []
    for dy in range(3):
        for dx in range(3):
            top = jnp.concatenate(
                [w2[dy, dx].T, jnp.zeros((n3, n5r), w2.dtype)], axis=1)
            bot = jnp.concatenate(
                [jnp.zeros((n5, n3r), w3.dtype), w3[dy, dx].T], axis=1)
            blocks.append(jnp.concatenate([top, bot], axis=0))
    wcT = jnp.concatenate(blocks, axis=1).astype(jnp.bfloat16)
    bcA = jnp.concatenate([b2, b3]).reshape(-1, 1)
    w4T = w4.T.astype(jnp.bfloat16)
    b4A = b4.reshape(-1, 1)
    cmask, pmask = _edge_masks(H, W)

    fn = functools.partial(_block_kernel, H=H, W=W, n1=n1, cr=cr)
    out = pl.pallas_call(
        fn,
        out_shape=jax.ShapeDtypeStruct((N, ctot, HW), jnp.float32),
        grid=(N,),
        in_specs=[
            pl.BlockSpec((1, Cin, HW), lambda n: (n, 0, 0)),
            pl.BlockSpec((n1 + cr, Cin), lambda n: (0, 0)),
            pl.BlockSpec((n1 + cr, 1), lambda n: (0, 0)),
            pl.BlockSpec((n3 + n5, 9 * cr), lambda n: (0, 0)),
            pl.BlockSpec((n3 + n5, 1), lambda n: (0, 0)),
            pl.BlockSpec((pc, Cin), lambda n: (0, 0)),
            pl.BlockSpec((pc, 1), lambda n: (0, 0)),
            pl.BlockSpec((9, 1, HW), lambda n: (0, 0, 0)),
            pl.BlockSpec((4, 1, HW), lambda n: (0, 0, 0)),
        ],
        out_specs=pl.BlockSpec((1, ctot, HW), lambda n: (n, 0, 0)),
        compiler_params=pltpu.CompilerParams(
            dimension_semantics=("parallel",)),
    )(x2, wrT, brA, wcT, bcA, w4T, b4A, cmask, pmask)
    return out.reshape(N, ctot, H, W)


# speedup vs baseline: 1.0583x; 1.0583x over previous
import functools

import jax
import jax.numpy as jnp
import numpy as np
from jax.experimental import pallas as pl
from jax.experimental.pallas import tpu as pltpu

_BLK = 2


def _block_kernel(x_ref, wr_ref, br_ref, wc_ref, bc_ref, w4_ref, b4_ref,
                  cmask_ref, pmask_ref, o_ref, *, H, W, n1, cr):
    for e in range(_BLK):
        xb = x_ref[e].astype(jnp.bfloat16)

        r = jax.lax.dot_general(
            wr_ref[...], xb, (((1,), (0,)), ((), ())),
            preferred_element_type=jnp.float32) + br_ref[...]
        y1 = r[0:n1]
        tb = r[n1:].astype(jnp.bfloat16)

        pieces = []
        k = 0
        for dy in (-1, 0, 1):
            for dx in (-1, 0, 1):
                o = dy * W + dx
                if o == 0:
                    pieces.append(tb)
                else:
                    pieces.append(jnp.roll(tb, -o, axis=1) * cmask_ref[k])
                k += 1
        col = jnp.concatenate(pieces, axis=0)

        y23 = jax.lax.dot_general(
            wc_ref[...], col, (((1,), (0,)), ((), ())),
            preferred_element_type=jnp.float32) + bc_ref[...]

        mw = jnp.maximum(xb, jnp.maximum(
            jnp.roll(xb, 1, axis=1) + pmask_ref[0],
            jnp.roll(xb, -1, axis=1) + pmask_ref[1]))
        m = jnp.maximum(mw, jnp.maximum(
            jnp.roll(mw, W, axis=1) + pmask_ref[2],
            jnp.roll(mw, -W, axis=1) + pmask_ref[3]))
        y4 = jax.lax.dot_general(
            w4_ref[...], m, (((1,), (0,)), ((), ())),
            preferred_element_type=jnp.float32) + b4_ref[...]

        o_ref[e] = jnp.concatenate([y1, y23, y4], axis=0)


def _edge_masks(H, W):
    p = np.arange(H * W)
    hc, wc = p // W, p % W
    cmasks = []
    for dy in (-1, 0, 1):
        for dx in (-1, 0, 1):
            valid = np.ones(H * W, bool)
            if dy < 0:
                valid &= hc >= 1
            if dy > 0:
                valid &= hc <= H - 2
            if dx < 0:
                valid &= wc >= 1
            if dx > 0:
                valid &= wc <= W - 2
            cmasks.append(valid.astype(np.float32))
    cmask = np.stack(cmasks).reshape(9, 1, H * W)
    pmask = np.stack([
        np.where(wc >= 1, 0.0, -np.inf),
        np.where(wc <= W - 2, 0.0, -np.inf),
        np.where(hc >= 1, 0.0, -np.inf),
        np.where(hc <= H - 2, 0.0, -np.inf),
    ]).reshape(4, 1, H * W)
    return (jnp.asarray(cmask, jnp.bfloat16), jnp.asarray(pmask, jnp.bfloat16))


def kernel(x, w1, b1, w2r, b2r, w2, b2, w3r, b3r, w3, b3, w4, b4):
    N, Cin, H, W = x.shape
    HW = H * W
    n1 = w1.shape[1]
    n3r, n3 = w2.shape[2], w2.shape[3]
    n5r, n5 = w3.shape[2], w3.shape[3]
    pc = w4.shape[1]
    cr = n3r + n5r
    ctot = n1 + n3 + n5 + pc

    x2 = x.reshape(N, Cin, HW)

    wrT = jnp.concatenate([w1, w2r, w3r], axis=1).T.astype(jnp.bfloat16)
    brA = jnp.concatenate([b1, b2r, b3r]).reshape(-1, 1)
    blocks = []
    for dy in range(3):
        for dx in range(3):
            top = jnp.concatenate(
                [w2[dy, dx].T, jnp.zeros((n3, n5r), w2.dtype)], axis=1)
            bot = jnp.concatenate(
                [jnp.zeros((n5, n3r), w3.dtype), w3[dy, dx].T], axis=1)
            blocks.append(jnp.concatenate([top, bot], axis=0))
    wcT = jnp.concatenate(blocks, axis=1).astype(jnp.bfloat16)
    bcA = jnp.concatenate([b2, b3]).reshape(-1, 1)
    w4T = w4.T.astype(jnp.bfloat16)
    b4A = b4.reshape(-1, 1)
    cmask, pmask = _edge_masks(H, W)

    fn = functools.partial(_block_kernel, H=H, W=W, n1=n1, cr=cr)
    out = pl.pallas_call(
        fn,
        out_shape=jax.ShapeDtypeStruct((N, ctot, HW), jnp.float32),
        grid=(N // _BLK,),
        in_specs=[
            pl.BlockSpec((_BLK, Cin, HW), lambda n: (n, 0, 0)),
            pl.BlockSpec((n1 + cr, Cin), lambda n: (0, 0)),
            pl.BlockSpec((n1 + cr, 1), lambda n: (0, 0)),
            pl.BlockSpec((n3 + n5, 9 * cr), lambda n: (0, 0)),
            pl.BlockSpec((n3 + n5, 1), lambda n: (0, 0)),
            pl.BlockSpec((pc, Cin), lambda n: (0, 0)),
            pl.BlockSpec((pc, 1), lambda n: (0, 0)),
            pl.BlockSpec((9, 1, HW), lambda n: (0, 0, 0)),
            pl.BlockSpec((4, 1, HW), lambda n: (0, 0, 0)),
        ],
        out_specs=pl.BlockSpec((_BLK, ctot, HW), lambda n: (n, 0, 0)),
        compiler_params=pltpu.CompilerParams(
            dimension_semantics=("parallel",)),
    )(x2, wrT, brA, wcT, bcA, w4T, b4A, cmask, pmask)
    return out.reshape(N, ctot, H, W)
